# Initial kernel scaffold; baseline (speedup 1.0000x reference)
import jax
import jax.numpy as jnp
from jax.experimental import pallas as pl


def kernel(hidden, mask, ln1_g, ln1_b, qkv_w, qkv_b, wo, bo, ln2_g, ln2_b, w1, b1, w2, b2):
    raise NotImplementedError("write your pallas kernel here")


# speedup vs baseline: 3.9090x; 3.9090x over previous
import jax
import jax.numpy as jnp
from jax.experimental import pallas as pl
from jax.experimental.pallas import tpu as pltpu

D = 768
NUM_HEADS = 12
HEAD_DIM = D // NUM_HEADS
ATT_SCALE = HEAD_DIM ** (-0.5)
FF = 3072
FF_TILE = 1536
N_LAYERS = 12
LN_EPS = 1e-5
BB = 16


def _ln(x, g, b):
    mu = jnp.mean(x, axis=-1, keepdims=True)
    var = jnp.mean(jnp.square(x - mu), axis=-1, keepdims=True)
    return (x - mu) * jax.lax.rsqrt(var + LN_EPS) * g + b


def _gelu_tanh(x):
    c = 0.7978845608028654
    return 0.5 * x * (1.0 + jnp.tanh(c * (x + 0.044715 * x * x * x)))


def _encoder_kernel(x_ref, mask_ref, ln1g_ref, ln1b_ref, qkvw_ref, qkvb_ref,
                    wo_ref, bo_ref, ln2g_ref, ln2b_ref, w1_ref, b1_ref,
                    w2_ref, b2_ref, out_ref):
    layer = pl.program_id(1)

    @pl.when(layer == 0)
    def _():
        out_ref[...] = x_ref[...]

    bb, s, _ = out_ref.shape
    x = out_ref[...].reshape(bb * s, D)

    xn = _ln(x, ln1g_ref[0], ln1b_ref[0]).astype(jnp.bfloat16)
    qkv = jnp.dot(xn, qkvw_ref[0],
                  preferred_element_type=jnp.float32) + qkvb_ref[0]
    qkv = qkv.reshape(bb, s, 3 * D)

    mask2d = mask_ref[0, 0]

    ctx_heads = []
    for h in range(NUM_HEADS):
        lo = h * HEAD_DIM
        qh = (qkv[:, :, lo:lo + HEAD_DIM] * ATT_SCALE).astype(jnp.bfloat16)
        kh = qkv[:, :, D + lo:D + lo + HEAD_DIM].astype(jnp.bfloat16)
        vh = qkv[:, :, 2 * D + lo:2 * D + lo + HEAD_DIM].astype(jnp.bfloat16)
        sc = jax.lax.dot_general(qh, kh, (((2,), (2,)), ((0,), (0,))),
                                 preferred_element_type=jnp.float32)
        sc = sc + mask2d[None]
        sc = sc - jnp.max(sc, axis=-1, keepdims=True)
        e = jnp.exp(sc)
        p = (e / jnp.sum(e, axis=-1, keepdims=True)).astype(jnp.bfloat16)
        ctx_heads.append(jax.lax.dot_general(
            p, vh, (((2,), (1,)), ((0,), (0,))),
            preferred_element_type=jnp.float32))
    ctx = jnp.concatenate(ctx_heads, axis=-1)
    ctx = ctx.reshape(bb * s, D).astype(jnp.bfloat16)

    attn = jnp.dot(ctx, wo_ref[0],
                   preferred_element_type=jnp.float32) + bo_ref[0]
    x = x + attn

    xn2 = _ln(x, ln2g_ref[0], ln2b_ref[0]).astype(jnp.bfloat16)
    acc = x + b2_ref[0]
    for t in range(FF // FF_TILE):
        fo = t * FF_TILE
        ht = jnp.dot(xn2, w1_ref[0][:, fo:fo + FF_TILE],
                     preferred_element_type=jnp.float32) + b1_ref[0][:, fo:fo + FF_TILE]
        ht = _gelu_tanh(ht).astype(jnp.bfloat16)
        acc = acc + jnp.dot(ht, w2_ref[0][fo:fo + FF_TILE, :],
                            preferred_element_type=jnp.float32)

    out_ref[...] = acc.reshape(bb, s, D)


def kernel(hidden, mask, ln1_g, ln1_b, qkv_w, qkv_b, wo, bo,
           ln2_g, ln2_b, w1, b1, w2, b2):
    B, S, _ = hidden.shape
    nb = B // BB

    qkv_w = qkv_w.astype(jnp.bfloat16)
    wo = wo.astype(jnp.bfloat16)
    w1 = w1.astype(jnp.bfloat16)
    w2 = w2.astype(jnp.bfloat16)

    return pl.pallas_call(
        _encoder_kernel,
        out_shape=jax.ShapeDtypeStruct((B, S, D), jnp.float32),
        grid_spec=pltpu.PrefetchScalarGridSpec(
            num_scalar_prefetch=0,
            grid=(nb, N_LAYERS),
            in_specs=[
                pl.BlockSpec((BB, S, D), lambda b, l: (b, 0, 0)),
                pl.BlockSpec((1, 1, S, S), lambda b, l: (0, 0, 0, 0)),
                pl.BlockSpec((1, 1, D), lambda b, l: (l, 0, 0)),
                pl.BlockSpec((1, 1, D), lambda b, l: (l, 0, 0)),
                pl.BlockSpec((1, D, 3 * D), lambda b, l: (l, 0, 0)),
                pl.BlockSpec((1, 1, 3 * D), lambda b, l: (l, 0, 0)),
                pl.BlockSpec((1, D, D), lambda b, l: (l, 0, 0)),
                pl.BlockSpec((1, 1, D), lambda b, l: (l, 0, 0)),
                pl.BlockSpec((1, 1, D), lambda b, l: (l, 0, 0)),
                pl.BlockSpec((1, 1, D), lambda b, l: (l, 0, 0)),
                pl.BlockSpec((1, D, FF), lambda b, l: (l, 0, 0)),
                pl.BlockSpec((1, 1, FF), lambda b, l: (l, 0, 0)),
                pl.BlockSpec((1, FF, D), lambda b, l: (l, 0, 0)),
                pl.BlockSpec((1, 1, D), lambda b, l: (l, 0, 0)),
            ],
            out_specs=pl.BlockSpec((BB, S, D), lambda b, l: (b, 0, 0)),
        ),
        compiler_params=pltpu.CompilerParams(
            dimension_semantics=("parallel", "arbitrary"),
            vmem_limit_bytes=56 * 1024 * 1024,
        ),
    )(hidden, mask, ln1_g, ln1_b, qkv_w, qkv_b, wo, bo,
      ln2_g, ln2_b, w1, b1, w2, b2)
